# baseline (speedup 1.0000x reference)
import functools
import numpy as np
import jax
import jax.numpy as jnp
from jax import lax
from jax.experimental import pallas as pl
from jax.experimental.pallas import tpu as pltpu
from jax.experimental.pallas import tpu_sc as plsc

KSEL = 8
RB = 8
CA = np.float32(10.0 * np.log2(np.e))
CB = np.float32(-10.0 * np.log2(np.log(2.0)))
C10 = np.float32(10.0)

SC_NC = 2
SC_NS = 16
SC_L = 16


def _tree(op, items):
    while len(items) > 1:
        nxt = [op(items[i], items[i + 1]) for i in range(0, len(items) - 1, 2)]
        if len(items) % 2:
            nxt.append(items[-1])
        items = nxt
    return items[0]


def _tc_body(logits_ref, u_ref, samples_ref, z_ref):
    _, kk, d = z_ref.shape
    ch = min(8192, d)
    nch = d // ch
    nsl = ch // 128
    tiny = np.float32(np.finfo(np.float32).tiny)
    neg_inf = np.float32(-np.inf)

    def phase_a(j, carry):
        ms, ss = carry
        js = pl.ds(j * ch, ch)
        new_ms, new_ss = [], []
        for r in range(RB):
            u = u_ref[r, :, js]
            wp = -jnp.log2(jnp.maximum(u, tiny))
            lp = CA * logits_ref[pl.ds(r, 1), js] + CB
            z = lp - C10 * jnp.log2(wp)
            z_ref[r, :, js] = z
            sl = [z[:, s * 128:(s + 1) * 128] for s in range(nsl)]
            cm = _tree(jnp.maximum, sl)
            mn = jnp.maximum(ms[r], cm)
            se = _tree(jnp.add, [jnp.exp2(x - mn) for x in sl])
            new_ss.append(ss[r] * jnp.exp2(ms[r] - mn) + se)
            new_ms.append(mn)
        return tuple(new_ms), tuple(new_ss)

    init_m = tuple(jnp.full((kk, 128), neg_inf, jnp.float32)
                   for _ in range(RB))
    init_s = tuple(jnp.zeros((kk, 128), jnp.float32) for _ in range(RB))
    ms, ss = jax.lax.fori_loop(0, nch, phase_a, (init_m, init_s))

    cs = []
    for r in range(RB):
        m = jnp.max(ms[r], axis=1, keepdims=True)
        s = jnp.sum(ss[r] * jnp.exp2(ms[r] - m), axis=1, keepdims=True)
        cs.append(m + jnp.log2(s))

    def phase_b(j, carry):
        js = pl.ds(j * ch, ch)
        rows = []
        for r in range(RB):
            y = z_ref[r, :, js] - cs[r]
            rows.append(jnp.max(y, axis=0, keepdims=True))
        stacked = jnp.concatenate(rows, axis=0)
        samples_ref[:, js] = jnp.exp2(stacked)
        return carry

    jax.lax.fori_loop(0, nch, phase_b, jnp.int32(0))


def _tc_build(B, K, d, interpret=False):
    return pl.pallas_call(
        _tc_body,
        grid=(B // RB,),
        in_specs=[
            pl.BlockSpec((RB, d), lambda i: (i, 0)),
            pl.BlockSpec((RB, K, d), lambda i: (i, 0, 0)),
        ],
        out_specs=pl.BlockSpec((RB, d), lambda i: (i, 0)),
        out_shape=jax.ShapeDtypeStruct((B, d), jnp.float32),
        scratch_shapes=[pltpu.VMEM((RB, K, d), jnp.float32)],
        compiler_params=pltpu.CompilerParams(
            dimension_semantics=("parallel",)),
        interpret=interpret,
    )


SC_S = 4


def _sc_body(logits_hbm, out_hbm, row_v, cand_v, shf_v):
    B, D = logits_hbm.shape
    L = SC_L
    S = SC_S
    nch = D // L
    npb = nch // S
    neg_inf = np.float32(-np.inf)
    wid = lax.axis_index("s") * 2 + lax.axis_index("c")
    for rr in range(B // 32):
        row = wid * (B // 32) + rr
        pltpu.sync_copy(logits_hbm.at[row], row_v)

        def bub(i, ts):
            out = []
            for s in range(S):
                x = row_v[pl.ds((i * S + s) * L, L)]
                cur = []
                for t in ts[s]:
                    hi = jnp.maximum(t, x)
                    x = jnp.minimum(t, x)
                    cur.append(hi)
                out.append(tuple(cur))
            return tuple(out)
        init = tuple(tuple(jnp.full((L,), neg_inf, jnp.float32)
                           for _ in range(KSEL)) for _ in range(S))
        ts = lax.fori_loop(0, npb, bub, init)

        idx = 0
        for s in range(S):
            for t in ts[s]:
                cand_v[pl.ds(idx * L, L)] = t
                idx += 1
        ncand = S * KSEL

        def redmax(v):
            m = v
            for sh in (8, 4, 2, 1):
                shf_v[pl.ds(0, L)] = m
                m = jnp.maximum(m, shf_v[pl.ds(sh, L)])
            return m[0]

        def redsum(v):
            m = v
            for sh in (8, 4, 2, 1):
                shf_v[pl.ds(0, L)] = m
                m = m + shf_v[pl.ds(sh, L)]
            return m[0]

        shf_v[pl.ds(L, L)] = jnp.full((L,), neg_inf, jnp.float32)

        def sel_iter(it, carry):
            t, thr, cum, done = carry
            m = jnp.full((L,), neg_inf, jnp.float32)
            for c in range(ncand):
                x = cand_v[pl.ds(c * L, L)]
                m = jnp.maximum(m, jnp.where(x < t, x, neg_inf))
            mx = redmax(m)
            cnt = jnp.zeros((L,), jnp.float32)
            for c in range(ncand):
                x = cand_v[pl.ds(c * L, L)]
                cnt = cnt + jnp.where(x == mx, 1.0, 0.0)
            shf_v[pl.ds(L, L)] = jnp.zeros((L,), jnp.float32)
            cum = cum + redsum(cnt)
            shf_v[pl.ds(L, L)] = jnp.full((L,), neg_inf, jnp.float32)
            hit = jnp.where(cum >= np.float32(KSEL), 1.0, 0.0)
            newly = hit * (1.0 - done)
            thr = jnp.where(newly > 0.0, mx, thr)
            done = jnp.maximum(done, hit)
            return (mx, thr, cum, done)

        init_s = (jnp.float32(np.inf), jnp.float32(0.0), jnp.float32(0.0),
                  jnp.float32(0.0))
        thr = lax.fori_loop(0, KSEL, sel_iter, init_s)[1]

        def p3(i, carry):
            js = pl.ds(i * L, L)
            row_v[js] = jnp.where(row_v[js] >= thr, 1.0, 0.0)
            return carry
        lax.fori_loop(0, nch, p3, jnp.int32(0))
        pltpu.sync_copy(row_v, out_hbm.at[row])


def _sc_build(B, d):
    mesh = plsc.VectorSubcoreMesh(core_axis_name="c", subcore_axis_name="s",
                                  num_cores=SC_NC, num_subcores=SC_NS)
    return functools.partial(
        pl.kernel,
        out_type=jax.ShapeDtypeStruct((B, d), jnp.float32),
        mesh=mesh,
        scratch_types=[
            pltpu.VMEM((d,), jnp.float32),
            pltpu.VMEM((SC_S * KSEL * SC_L,), jnp.float32),
            pltpu.VMEM((2 * SC_L,), jnp.float32),
        ],
    )(_sc_body)


def kernel(logits, uniform):
    B, d = logits.shape
    K = uniform.shape[1]
    discrete = _sc_build(B, d)(logits)
    samples = _tc_build(B, K, d)(logits, uniform)
    return samples, discrete
